# baseline (speedup 1.0000x reference)
import functools

import jax
import jax.numpy as jnp
from jax import lax
from jax.experimental import pallas as pl
from jax.experimental.pallas import tpu as pltpu
from jax.experimental.pallas import tpu_sc as plsc

NUM_CORES = 2
NUM_SUBCORES = 16
NW = NUM_CORES * NUM_SUBCORES

B = 16384 * 26
D = 64
B_PER_W = B // NW
CHUNK = 512
N_CHUNKS = B_PER_W // CHUNK

_mesh = plsc.VectorSubcoreMesh(core_axis_name="c", subcore_axis_name="s")


@functools.partial(
    pl.kernel,
    mesh=_mesh,
    compiler_params=pltpu.CompilerParams(use_tc_tiling_on_sc=False),
    out_type=jax.ShapeDtypeStruct((B, D), jnp.float32),
    scratch_types=[
        pltpu.VMEM((CHUNK,), jnp.int32),
        pltpu.VMEM((CHUNK, D), jnp.float32),
        pltpu.SemaphoreType.DMA,
    ],
)
def _gather_kernel(idx_hbm, table_hbm, out_hbm, idx_v, rows_v, sem):
    wid = lax.axis_index("s") * NUM_CORES + lax.axis_index("c")
    base0 = wid * B_PER_W

    def body(i, carry):
        base = base0 + i * CHUNK
        pltpu.sync_copy(idx_hbm.at[pl.ds(base, CHUNK)], idx_v)
        pltpu.async_copy(table_hbm.at[idx_v], rows_v, sem).wait()
        pltpu.sync_copy(rows_v, out_hbm.at[pl.ds(base, CHUNK)])
        return carry

    lax.fori_loop(0, N_CHUNKS, body, 0)


def kernel(token_ids, weight):
    flat = token_ids.reshape(-1).astype(jnp.int32)
    out = _gather_kernel(flat, weight)
    return out.reshape(token_ids.shape + (weight.shape[1],))


# speedup vs baseline: 1.0238x; 1.0238x over previous
import functools

import jax
import jax.numpy as jnp
from jax import lax
from jax.experimental import pallas as pl
from jax.experimental.pallas import tpu as pltpu
from jax.experimental.pallas import tpu_sc as plsc

NUM_CORES = 2
NUM_SUBCORES = 16
NW = NUM_CORES * NUM_SUBCORES

B = 16384 * 26
D = 64
B_PER_W = B // NW
CHUNK = 832
N_CHUNKS = B_PER_W // CHUNK
NBUF = 2

_mesh = plsc.VectorSubcoreMesh(core_axis_name="c", subcore_axis_name="s")


@functools.partial(
    pl.kernel,
    mesh=_mesh,
    compiler_params=pltpu.CompilerParams(use_tc_tiling_on_sc=False),
    out_type=jax.ShapeDtypeStruct((B, D), jnp.float32),
    scratch_types=[
        pltpu.VMEM((NBUF, CHUNK), jnp.int32),
        pltpu.VMEM((NBUF, CHUNK, D), jnp.float32),
        pltpu.SemaphoreType.DMA,
        pltpu.SemaphoreType.DMA,
        pltpu.SemaphoreType.DMA,
        pltpu.SemaphoreType.DMA,
        pltpu.SemaphoreType.DMA,
        pltpu.SemaphoreType.DMA,
    ],
)
def _gather_kernel(idx_hbm, table_hbm, out_hbm, idx_v, rows_v,
                   sem_i0, sem_i1, sem_g0, sem_g1, sem_o0, sem_o1):
    wid = lax.axis_index("s") * NUM_CORES + lax.axis_index("c")
    base0 = wid * B_PER_W
    sem_i = (sem_i0, sem_i1)
    sem_g = (sem_g0, sem_g1)
    sem_o = (sem_o0, sem_o1)

    for b in range(NBUF):
        pltpu.async_copy(idx_hbm.at[pl.ds(base0 + b * CHUNK, CHUNK)],
                         idx_v.at[b], sem_i[b])

    def body(j, carry):
        for b in range(NBUF):
            i = NBUF * j + b
            base = base0 + i * CHUNK

            @pl.when(j > 0)
            def _wait_out():
                pltpu.make_async_copy(rows_v.at[b],
                                      out_hbm.at[pl.ds(base0, CHUNK)],
                                      sem_o[b]).wait()

            pltpu.make_async_copy(idx_hbm.at[pl.ds(base0, CHUNK)],
                                  idx_v.at[b], sem_i[b]).wait()

            gcp = pltpu.async_copy(table_hbm.at[idx_v.at[b]],
                                   rows_v.at[b], sem_g[b])
            gcp.wait()

            @pl.when(i + NBUF < N_CHUNKS)
            def _prefetch():
                pltpu.async_copy(idx_hbm.at[pl.ds(base + NBUF * CHUNK, CHUNK)],
                                 idx_v.at[b], sem_i[b])

            pltpu.async_copy(rows_v.at[b], out_hbm.at[pl.ds(base, CHUNK)],
                             sem_o[b])
        return carry

    lax.fori_loop(0, N_CHUNKS // NBUF, body, 0)

    for b in range(NBUF):
        pltpu.make_async_copy(rows_v.at[b], out_hbm.at[pl.ds(base0, CHUNK)],
                              sem_o[b]).wait()


def kernel(token_ids, weight):
    flat = token_ids.reshape(-1).astype(jnp.int32)
    out = _gather_kernel(flat, weight)
    return out.reshape(token_ids.shape + (weight.shape[1],))


# speedup vs baseline: 1.0266x; 1.0027x over previous
import functools

import jax
import jax.numpy as jnp
from jax import lax
from jax.experimental import pallas as pl
from jax.experimental.pallas import tpu as pltpu
from jax.experimental.pallas import tpu_sc as plsc

NUM_CORES = 2
NUM_SUBCORES = 16
NW = NUM_CORES * NUM_SUBCORES

S = 16384
C = 26
D = 64
S_CHUNK = S // NW
NBUF = 2

_mesh = plsc.VectorSubcoreMesh(core_axis_name="c", subcore_axis_name="s")


@functools.partial(
    pl.kernel,
    mesh=_mesh,
    compiler_params=pltpu.CompilerParams(use_tc_tiling_on_sc=False),
    out_type=jax.ShapeDtypeStruct((S, C, D), jnp.float32),
    scratch_types=[
        pltpu.VMEM((NBUF, S_CHUNK), jnp.int32),
        pltpu.VMEM((NBUF, S_CHUNK, D), jnp.float32),
        pltpu.SemaphoreType.DMA,
        pltpu.SemaphoreType.DMA,
        pltpu.SemaphoreType.DMA,
        pltpu.SemaphoreType.DMA,
        pltpu.SemaphoreType.DMA,
        pltpu.SemaphoreType.DMA,
    ],
)
def _gather_kernel(idx_hbm, table_hbm, out_hbm, idx_v, rows_v,
                   sem_i0, sem_i1, sem_g0, sem_g1, sem_o0, sem_o1):
    wid = lax.axis_index("s") * NUM_CORES + lax.axis_index("c")
    s0 = wid * S_CHUNK
    sem_i = (sem_i0, sem_i1)
    sem_g = (sem_g0, sem_g1)
    sem_o = (sem_o0, sem_o1)

    for b in range(NBUF):
        pltpu.async_copy(idx_hbm.at[b, pl.ds(s0, S_CHUNK)],
                         idx_v.at[b], sem_i[b])

    def body(j, carry):
        for b in range(NBUF):
            col = NBUF * j + b

            @pl.when(j > 0)
            def _wait_out():
                pltpu.make_async_copy(rows_v.at[b],
                                      out_hbm.at[pl.ds(s0, S_CHUNK), 0],
                                      sem_o[b]).wait()

            pltpu.make_async_copy(idx_hbm.at[0, pl.ds(s0, S_CHUNK)],
                                  idx_v.at[b], sem_i[b]).wait()

            pltpu.async_copy(table_hbm.at[idx_v.at[b]],
                             rows_v.at[b], sem_g[b]).wait()

            @pl.when(col + NBUF < C)
            def _prefetch():
                pltpu.async_copy(idx_hbm.at[col + NBUF, pl.ds(s0, S_CHUNK)],
                                 idx_v.at[b], sem_i[b])

            pltpu.async_copy(rows_v.at[b],
                             out_hbm.at[pl.ds(s0, S_CHUNK), col],
                             sem_o[b])
        return carry

    lax.fori_loop(0, C // NBUF, body, 0)

    for b in range(NBUF):
        pltpu.make_async_copy(rows_v.at[b],
                              out_hbm.at[pl.ds(s0, S_CHUNK), 0],
                              sem_o[b]).wait()


def kernel(token_ids, weight):
    return _gather_kernel(token_ids.T, weight)


# speedup vs baseline: 1.0726x; 1.0449x over previous
import functools

import jax
import jax.numpy as jnp
from jax import lax
from jax.experimental import pallas as pl
from jax.experimental.pallas import tpu as pltpu
from jax.experimental.pallas import tpu_sc as plsc

NUM_CORES = 2
NUM_SUBCORES = 16
NW = NUM_CORES * NUM_SUBCORES

S = 16384
C = 26
C_PAD = 32
D = 64
S_CHUNK = S // NW
NBUF = 2

_mesh = plsc.VectorSubcoreMesh(core_axis_name="c", subcore_axis_name="s")


@functools.partial(
    pl.kernel,
    mesh=_mesh,
    compiler_params=pltpu.CompilerParams(use_tc_tiling_on_sc=False),
    out_type=jax.ShapeDtypeStruct((C, S, D), jnp.float32),
    scratch_types=[
        pltpu.VMEM((NBUF, S_CHUNK), jnp.int32),
        pltpu.VMEM((NBUF, S_CHUNK, D), jnp.float32),
        pltpu.SemaphoreType.DMA,
        pltpu.SemaphoreType.DMA,
        pltpu.SemaphoreType.DMA,
        pltpu.SemaphoreType.DMA,
        pltpu.SemaphoreType.DMA,
        pltpu.SemaphoreType.DMA,
    ],
)
def _gather_kernel(idx_hbm, table_hbm, out_hbm, idx_v, rows_v,
                   sem_i0, sem_i1, sem_g0, sem_g1, sem_o0, sem_o1):
    wid = lax.axis_index("s") * NUM_CORES + lax.axis_index("c")
    s0 = wid * S_CHUNK
    sem_i = (sem_i0, sem_i1)
    sem_g = (sem_g0, sem_g1)
    sem_o = (sem_o0, sem_o1)

    for b in range(NBUF):
        pltpu.async_copy(idx_hbm.at[b, pl.ds(s0, S_CHUNK)],
                         idx_v.at[b], sem_i[b])

    def body(j, carry):
        for b in range(NBUF):
            col = NBUF * j + b

            @pl.when(j > 0)
            def _wait_out():
                pltpu.make_async_copy(rows_v.at[b],
                                      out_hbm.at[0, pl.ds(s0, S_CHUNK)],
                                      sem_o[b]).wait()

            pltpu.make_async_copy(idx_hbm.at[0, pl.ds(s0, S_CHUNK)],
                                  idx_v.at[b], sem_i[b]).wait()

            pltpu.async_copy(table_hbm.at[idx_v.at[b]],
                             rows_v.at[b], sem_g[b]).wait()

            @pl.when(col + NBUF < C)
            def _prefetch():
                pltpu.async_copy(idx_hbm.at[col + NBUF, pl.ds(s0, S_CHUNK)],
                                 idx_v.at[b], sem_i[b])

            pltpu.async_copy(rows_v.at[b],
                             out_hbm.at[col, pl.ds(s0, S_CHUNK)],
                             sem_o[b])
        return carry

    lax.fori_loop(0, C // NBUF, body, 0)

    for b in range(NBUF):
        pltpu.make_async_copy(rows_v.at[b],
                              out_hbm.at[0, pl.ds(s0, S_CHUNK)],
                              sem_o[b]).wait()


def kernel(token_ids, weight):
    idx = jnp.concatenate(
        [token_ids.T, jnp.zeros((C_PAD - C, S), token_ids.dtype)], axis=0)
    out = _gather_kernel(idx, weight)
    return jnp.transpose(out, (1, 0, 2))


# speedup vs baseline: 1.1442x; 1.0668x over previous
import functools

import jax
import jax.numpy as jnp
from jax import lax
from jax.experimental import pallas as pl
from jax.experimental.pallas import tpu as pltpu
from jax.experimental.pallas import tpu_sc as plsc

NUM_CORES = 2
NUM_SUBCORES = 16
NW = NUM_CORES * NUM_SUBCORES

S = 16384
C = 26
C_PAD = 32
D = 64
V = 1_000_000
S_CHUNK = S // NW
NBUF = 2
LANES = 16

_mesh = plsc.VectorSubcoreMesh(core_axis_name="c", subcore_axis_name="s")


@functools.partial(
    pl.kernel,
    mesh=_mesh,
    compiler_params=pltpu.CompilerParams(use_tc_tiling_on_sc=False),
    out_type=jax.ShapeDtypeStruct((C, S, D), jnp.float32),
    scratch_types=[
        pltpu.VMEM((NBUF, S_CHUNK), jnp.int32),
        pltpu.VMEM((NBUF, S_CHUNK), jnp.int32),
        pltpu.VMEM((NBUF, S_CHUNK, D), jnp.float32),
        pltpu.SemaphoreType.DMA,
        pltpu.SemaphoreType.DMA,
        pltpu.SemaphoreType.DMA,
        pltpu.SemaphoreType.DMA,
        pltpu.SemaphoreType.DMA,
        pltpu.SemaphoreType.DMA,
    ],
)
def _gather_kernel(idx_hbm, table_hbm, out_hbm, idx_v, idx2_v, rows_v,
                   sem_i0, sem_i1, sem_g0, sem_g1, sem_o0, sem_o1):
    wid = lax.axis_index("s") * NUM_CORES + lax.axis_index("c")
    s0 = wid * S_CHUNK
    sem_i = (sem_i0, sem_i1)
    sem_g = (sem_g0, sem_g1)
    sem_o = (sem_o0, sem_o1)

    for b in range(NBUF):
        pltpu.async_copy(idx_hbm.at[b, pl.ds(s0, S_CHUNK)],
                         idx_v.at[b], sem_i[b])

    def body(j, carry):
        for b in range(NBUF):
            col = NBUF * j + b

            @pl.when(j > 0)
            def _wait_out():
                pltpu.make_async_copy(rows_v.at[b],
                                      out_hbm.at[0, pl.ds(s0, S_CHUNK)],
                                      sem_o[b]).wait()

            pltpu.make_async_copy(idx_hbm.at[0, pl.ds(s0, S_CHUNK)],
                                  idx_v.at[b], sem_i[b]).wait()

            def dbl(k, carry2):
                sl = pl.ds(k * LANES, LANES)
                idx2_v[b, sl] = idx_v[b, sl] * 2
                return carry2
            lax.fori_loop(0, S_CHUNK // LANES, dbl, 0)

            pltpu.async_copy(table_hbm.at[idx2_v.at[b]],
                             rows_v.at[b], sem_g[b]).wait()

            @pl.when(col + NBUF < C)
            def _prefetch():
                pltpu.async_copy(idx_hbm.at[col + NBUF, pl.ds(s0, S_CHUNK)],
                                 idx_v.at[b], sem_i[b])

            pltpu.async_copy(rows_v.at[b],
                             out_hbm.at[col, pl.ds(s0, S_CHUNK)],
                             sem_o[b])
        return carry

    lax.fori_loop(0, C // NBUF, body, 0)

    for b in range(NBUF):
        pltpu.make_async_copy(rows_v.at[b],
                              out_hbm.at[0, pl.ds(s0, S_CHUNK)],
                              sem_o[b]).wait()


def kernel(token_ids, weight):
    idx = jnp.concatenate(
        [token_ids.T, jnp.zeros((C_PAD - C, S), token_ids.dtype)], axis=0)
    wpad = jnp.pad(weight, ((0, 0), (0, 64)))
    w2 = wpad.reshape(2 * V, D)
    out = _gather_kernel(idx, w2)
    return jnp.transpose(out, (1, 0, 2))
